# baseline (speedup 1.0000x reference)
import functools

import jax
import jax.numpy as jnp
from jax import lax
from jax.experimental import pallas as pl
from jax.experimental.pallas import tpu as pltpu
from jax.experimental.pallas import tpu_sc as plsc

EMB_D = 300
A_D = 256
B_OFF = 172
B_D = 128
NUM_WORKERS = 32
CHUNK = 64
XCOLS = 50


def _sc_gather(x, table, table_b):
    n_rows = x.shape[0]
    B = n_rows * XCOLS
    b_per_w = B // NUM_WORKERS
    xrows_per_w = n_rows // NUM_WORKERS
    n_chunks = b_per_w // CHUNK
    mesh = plsc.VectorSubcoreMesh(core_axis_name="c", subcore_axis_name="s")

    @functools.partial(
        pl.kernel,
        mesh=mesh,
        out_type=jax.ShapeDtypeStruct((B, EMB_D), jnp.float32),
        scratch_types=[
            pltpu.VMEM((xrows_per_w, XCOLS), jnp.int32),
            pltpu.VMEM((b_per_w,), jnp.int32),
            pltpu.VMEM((CHUNK, EMB_D), jnp.float32),
            pltpu.VMEM((CHUNK, EMB_D), jnp.float32),
            pltpu.VMEM((CHUNK, B_D), jnp.float32),
            pltpu.VMEM((CHUNK, B_D), jnp.float32),
            pltpu.SemaphoreType.DMA,
            pltpu.SemaphoreType.DMA,
            pltpu.SemaphoreType.DMA,
            pltpu.SemaphoreType.DMA,
        ],
        compiler_params=pltpu.CompilerParams(needs_layout_passes=False),
    )
    def k(x_hbm, t_hbm, tb_hbm, out_hbm, x_v, idx_v,
          ob0, ob1, bb0, bb1, gs0, gs1, ws0, ws1):
        wid = lax.axis_index("s") * 2 + lax.axis_index("c")
        base0 = wid * b_per_w
        pltpu.sync_copy(x_hbm.at[pl.ds(wid * xrows_per_w, xrows_per_w)], x_v)

        cols = lax.iota(jnp.int32, 16)
        ta_hbm = t_hbm.at[:, pl.ds(0, A_D)]

        def crow(r, carry):
            rs = jnp.full((16,), r, jnp.int32)
            for off in (0, 16, 32, 34):
                v = plsc.load_gather(x_v, [rs, cols + off])
                plsc.store_scatter(idx_v, [r * XCOLS + off + cols], v)
            return carry

        lax.fori_loop(0, xrows_per_w, crow, 0)

        def start_gather(g, obuf, bbuf, gsem):
            isl = idx_v.at[pl.ds(g * CHUNK, CHUNK)]
            pltpu.async_copy(ta_hbm.at[isl], obuf.at[:, pl.ds(0, A_D)], gsem)
            pltpu.async_copy(tb_hbm.at[isl], bbuf, gsem)

        def wait_gather(g, obuf, bbuf, gsem):
            isl = idx_v.at[pl.ds(g * CHUNK, CHUNK)]
            pltpu.make_async_copy(ta_hbm.at[isl],
                                  obuf.at[:, pl.ds(0, A_D)], gsem).wait()
            pltpu.make_async_copy(tb_hbm.at[isl], bbuf, gsem).wait()

        def start_write(g, obuf, wsem):
            pltpu.async_copy(obuf, out_hbm.at[pl.ds(base0 + g * CHUNK, CHUNK)],
                             wsem)

        def wait_write(g, obuf, wsem):
            pltpu.make_async_copy(obuf,
                                  out_hbm.at[pl.ds(base0 + g * CHUNK, CHUNK)],
                                  wsem).wait()

        def merge(obuf, bbuf):
            def row(r, carry):
                rs = jnp.full((16,), r, jnp.int32)
                for lo, so in ((84, 256), (100, 272), (112, 284)):
                    v = plsc.load_gather(bbuf, [rs, cols + lo])
                    plsc.store_scatter(obuf, [rs, cols + so], v)
                return carry
            lax.fori_loop(0, CHUNK, row, 0)

        start_gather(0, ob0, bb0, gs0)

        def body(i, carry):
            g0 = i * 2
            g1 = g0 + 1
            wait_gather(g0, ob0, bb0, gs0)

            @pl.when(i > 0)
            def _():
                wait_write(g0 - 1, ob1, ws1)

            start_gather(g1, ob1, bb1, gs1)
            merge(ob0, bb0)
            start_write(g0, ob0, ws0)
            wait_gather(g1, ob1, bb1, gs1)
            wait_write(g0, ob0, ws0)

            @pl.when(g1 + 1 < n_chunks)
            def _():
                start_gather(g1 + 1, ob0, bb0, gs0)

            merge(ob1, bb1)
            start_write(g1, ob1, ws1)
            return carry

        lax.fori_loop(0, n_chunks // 2, body, 0)
        wait_write(n_chunks - 1, ob1, ws1)

    return k(x, table, table_b)


def kernel(x, table):
    xi = x.astype(jnp.int32)
    table_b = table[:, B_OFF:B_OFF + B_D] + jnp.float32(0.0)
    out = _sc_gather(xi, table, table_b)
    return out.reshape(x.shape[0], x.shape[1], EMB_D)


# speedup vs baseline: 1.3183x; 1.3183x over previous
import functools

import jax
import jax.numpy as jnp
from jax import lax
from jax.experimental import pallas as pl
from jax.experimental.pallas import tpu as pltpu
from jax.experimental.pallas import tpu_sc as plsc

EMB_D = 300
A_D = 256
B_OFF = 172
B_D = 128
NUM_WORKERS = 32
XCOLS = 50
XPAD = 56


def _sc_gather(x, table, table_b):
    n_rows = x.shape[0]
    rows_per_w = n_rows // NUM_WORKERS
    mesh = plsc.VectorSubcoreMesh(core_axis_name="c", subcore_axis_name="s")

    @functools.partial(
        pl.kernel,
        mesh=mesh,
        out_type=jax.ShapeDtypeStruct((n_rows, XCOLS, EMB_D), jnp.float32),
        scratch_types=[
            pltpu.VMEM((rows_per_w, XCOLS), jnp.int32),
            pltpu.VMEM((rows_per_w * XPAD,), jnp.int32),
            pltpu.VMEM((XCOLS, EMB_D), jnp.float32),
            pltpu.VMEM((XCOLS, EMB_D), jnp.float32),
            pltpu.VMEM((XCOLS, B_D), jnp.float32),
            pltpu.VMEM((XCOLS, B_D), jnp.float32),
            pltpu.SemaphoreType.DMA,
            pltpu.SemaphoreType.DMA,
            pltpu.SemaphoreType.DMA,
            pltpu.SemaphoreType.DMA,
        ],
        compiler_params=pltpu.CompilerParams(needs_layout_passes=False),
    )
    def k(x_hbm, t_hbm, tb_hbm, out_hbm, x_v, idx_v,
          ob0, ob1, bb0, bb1, gs0, gs1, ws0, ws1):
        wid = lax.axis_index("s") * 2 + lax.axis_index("c")
        base0 = wid * rows_per_w
        pltpu.sync_copy(x_hbm.at[pl.ds(base0, rows_per_w)], x_v)

        cols = lax.iota(jnp.int32, 16)
        ta_hbm = t_hbm.at[:, pl.ds(0, A_D)]

        def crow(r, carry):
            rs = jnp.full((16,), r, jnp.int32)
            for off in (0, 16, 32, 34):
                v = plsc.load_gather(x_v, [rs, cols + off])
                plsc.store_scatter(idx_v, [r * XPAD + off + cols], v)
            return carry

        lax.fori_loop(0, rows_per_w, crow, 0)

        def start_gather(e, obuf, bbuf, gsem):
            isl = idx_v.at[pl.ds(e * XPAD, XCOLS)]
            pltpu.async_copy(ta_hbm.at[isl], obuf.at[:, pl.ds(0, A_D)], gsem)
            pltpu.async_copy(tb_hbm.at[isl], bbuf, gsem)

        def wait_gather(e, obuf, bbuf, gsem):
            isl = idx_v.at[pl.ds(e * XPAD, XCOLS)]
            pltpu.make_async_copy(ta_hbm.at[isl],
                                  obuf.at[:, pl.ds(0, A_D)], gsem).wait()
            pltpu.make_async_copy(tb_hbm.at[isl], bbuf, gsem).wait()

        def start_write(e, obuf, wsem):
            pltpu.async_copy(obuf, out_hbm.at[base0 + e], wsem)

        def wait_write(e, obuf, wsem):
            pltpu.make_async_copy(obuf, out_hbm.at[base0 + e], wsem).wait()

        def merge(obuf, bbuf):
            def row(r, carry):
                rs = jnp.full((16,), r, jnp.int32)
                for lo, so in ((84, 256), (100, 272), (112, 284)):
                    v = plsc.load_gather(bbuf, [rs, cols + lo])
                    plsc.store_scatter(obuf, [rs, cols + so], v)
                return carry
            lax.fori_loop(0, XCOLS, row, 0)

        start_gather(0, ob0, bb0, gs0)

        def body(i, carry):
            e0 = i * 2
            e1 = e0 + 1
            wait_gather(e0, ob0, bb0, gs0)

            @pl.when(i > 0)
            def _():
                wait_write(e0 - 1, ob1, ws1)

            start_gather(e1, ob1, bb1, gs1)
            merge(ob0, bb0)
            start_write(e0, ob0, ws0)
            wait_gather(e1, ob1, bb1, gs1)
            wait_write(e0, ob0, ws0)

            @pl.when(e1 + 1 < rows_per_w)
            def _():
                start_gather(e1 + 1, ob0, bb0, gs0)

            merge(ob1, bb1)
            start_write(e1, ob1, ws1)
            return carry

        lax.fori_loop(0, rows_per_w // 2, body, 0)
        wait_write(rows_per_w - 1, ob1, ws1)

    return k(x, table, table_b)


def kernel(x, table):
    xi = x.astype(jnp.int32)
    table_b = table[:, B_OFF:B_OFF + B_D] + jnp.float32(0.0)
    return _sc_gather(xi, table, table_b)
